# baseline (speedup 1.0000x reference)
import functools

import jax
import jax.numpy as jnp
import numpy as np
from jax import lax
from jax.experimental import pallas as pl
from jax.experimental.pallas import tpu as pltpu
from jax.experimental.pallas import tpu_sc as plsc

B = 4096
N = 64
OUT = N * (N + 1) // 2

_NC = 2
_NS = 16
_NW = _NC * _NS
_U = 8
_UNITS = OUT // _U
_MAXT = -(-_UNITS // _NW)


def _triu_m() -> np.ndarray:
    rows, cols = np.triu_indices(N)
    return (rows * N + cols).astype(np.int32)


_NBUF = 3


def _body(in_hbm, idx_hbm, out_hbm, vidx, stage,
          semg0, semg1, semg2, semw0, semw1, semw2):
    wid = lax.axis_index("s") * _NC + lax.axis_index("c")
    semg = (semg0, semg1, semg2)
    semw = (semw0, semw1, semw2)
    pltpu.sync_copy(idx_hbm, vidx)

    def _guarded(t, fn):
        u = wid + t * _NW

        @pl.when(u < _UNITS)
        def _():
            fn(pl.multiple_of(u * _U, _U), t % _NBUF)

    def _start_g(row0, b):
        pltpu.async_copy(in_hbm.at[vidx.at[pl.ds(row0, _U)]],
                         stage.at[b], semg[b])

    def _wait_g(row0, b):
        pltpu.make_async_copy(in_hbm.at[vidx.at[pl.ds(row0, _U)]],
                              stage.at[b], semg[b]).wait()

    def _start_w(row0, b):
        pltpu.async_copy(stage.at[b], out_hbm.at[pl.ds(row0, _U)], semw[b])

    def _wait_w(row0, b):
        pltpu.make_async_copy(stage.at[b], out_hbm.at[pl.ds(row0, _U)],
                              semw[b]).wait()

    for t in range(_MAXT + 1):
        if t < _MAXT:
            if t >= _NBUF:
                _guarded(t - _NBUF, _wait_w)
            _guarded(t, _start_g)
        if t >= 1:
            _guarded(t - 1, _wait_g)
            _guarded(t - 1, _start_w)
    for t in range(max(0, _MAXT - _NBUF), _MAXT):
        _guarded(t, _wait_w)


def kernel(input):
    mesh = plsc.VectorSubcoreMesh(core_axis_name="c", subcore_axis_name="s")
    k = functools.partial(
        pl.kernel,
        out_type=jax.ShapeDtypeStruct((OUT, B), jnp.float32),
        mesh=mesh,
        scratch_types=[
            pltpu.VMEM((OUT,), jnp.int32),
            pltpu.VMEM((_NBUF, _U, B), jnp.float32),
            pltpu.SemaphoreType.DMA,
            pltpu.SemaphoreType.DMA,
            pltpu.SemaphoreType.DMA,
            pltpu.SemaphoreType.DMA,
            pltpu.SemaphoreType.DMA,
            pltpu.SemaphoreType.DMA,
        ],
        compiler_params=pltpu.CompilerParams(use_tc_tiling_on_sc=True,
                                             needs_layout_passes=False),
    )(_body)
    in_t = input.transpose(1, 2, 0).reshape(N * N, B)
    out_t = k(in_t, jnp.asarray(_triu_m()))
    return out_t.T
